# baseline (speedup 1.0000x reference)
import functools

import jax
import jax.numpy as jnp
from jax import lax
from jax.experimental import pallas as pl
from jax.experimental.pallas import tpu as pltpu
from jax.experimental.pallas import tpu_sc as plsc

N = 8192
DIM = 4096
K = 3

SC_ROWS = 2048
TC_ROWS = N - SC_ROWS
ROWS_PER_BLOCK = 1024
TC_GRID = TC_ROWS // ROWS_PER_BLOCK
SUBL = ROWS_PER_BLOCK // 128

NW = 32
RPT = SC_ROWS // NW
GROUP = 8
NGROUPS = RPT // GROUP
LANES = 16


def _norms_body(h_ref, out_ref):
    x = h_ref[...]
    s = jnp.sum(x * x, axis=1)
    out_ref[...] = s.reshape(SUBL, 128)


_UNROLL = 8


def _row_sumsq(buf_ref, r):
    def step(i, acc):
        base = i * (LANES * _UNROLL)
        for u in range(_UNROLL):
            x = buf_ref[r, pl.ds(base + u * LANES, LANES)]
            acc = acc + x * x
        return acc

    acc = lax.fori_loop(0, DIM // (LANES * _UNROLL), step,
                        jnp.zeros((LANES,), jnp.float32))
    return jnp.sum(acc)


def _sc_norms_body(hid_hbm, out_hbm, buf0, buf1, stage, sem0, sem1):
    wid = lax.axis_index("s") * 2 + lax.axis_index("c")
    base = TC_ROWS + wid * RPT
    bufs = (buf0, buf1)
    sems = (sem0, sem1)

    def start_group(g):
        b = g % 2
        cp = pltpu.make_async_copy(
            hid_hbm.at[pl.ds(base + g * GROUP, GROUP), :], bufs[b], sems[b])
        cp.start()
        return cp

    lane = lax.iota(jnp.int32, LANES)
    pending = start_group(0)
    vec = jnp.zeros((LANES,), jnp.float32)
    for g in range(NGROUPS):
        nxt = start_group(g + 1) if g + 1 < NGROUPS else None
        pending.wait()
        for r in range(GROUP):
            sv = _row_sumsq(bufs[g % 2], r)
            j = (g * GROUP + r) % LANES
            vec = jnp.where(lane == j, sv, vec)
        if (g * GROUP + GROUP) % LANES == 0:
            k16 = g * GROUP + GROUP - LANES
            stage[pl.ds(k16, LANES)] = vec
        pending = nxt

    pltpu.sync_copy(stage, out_hbm.at[pl.ds(wid * RPT, RPT)])


_sc_norms = functools.partial(
    pl.kernel,
    mesh=plsc.VectorSubcoreMesh(core_axis_name="c", subcore_axis_name="s"),
    out_type=jax.ShapeDtypeStruct((SC_ROWS,), jnp.float32),
    scratch_types=[
        pltpu.VMEM((GROUP, DIM), jnp.float32),
        pltpu.VMEM((GROUP, DIM), jnp.float32),
        pltpu.VMEM((RPT,), jnp.float32),
        pltpu.SemaphoreType.DMA,
        pltpu.SemaphoreType.DMA,
    ],
    compiler_params=pltpu.CompilerParams(needs_layout_passes=False),
    cost_estimate=pl.CostEstimate(
        flops=2 * SC_ROWS * DIM,
        bytes_accessed=SC_ROWS * DIM * 4,
        transcendentals=0,
    ),
)(_sc_norms_body)


def _select_body(norms_ref, tokens_ref, hid_ref, anchors_ref, meta_ref, sem):
    v = norms_ref[...]
    row = lax.broadcasted_iota(jnp.int32, v.shape, 0)
    lane = lax.broadcasted_iota(jnp.int32, v.shape, 1)
    gidx = row * 128 + lane
    big = jnp.int32(2**31 - 1)

    idxs = []
    for _ in range(K):
        m = jnp.max(v)
        cand = jnp.where(v == m, gidx, big)
        ik = jnp.min(cand)
        idxs.append(ik)
        v = jnp.where(gidx == ik, jnp.float32(-1.0), v)

    t = tokens_ref[...]
    toks = [jnp.sum(jnp.where(gidx == ik, t, 0)) for ik in idxs]

    lane8 = lax.broadcasted_iota(jnp.int32, (8, 128), 1)
    meta = jnp.where(lane8 == 0, toks[0],
                     jnp.where(lane8 == 1, toks[1],
                               jnp.where(lane8 == 2, toks[2], 0)))
    meta_ref[...] = meta

    for k, ik in enumerate(idxs):
        cp = pltpu.make_async_copy(hid_ref.at[pl.ds(ik, 1), :],
                                   anchors_ref.at[pl.ds(k, 1), :], sem)
        cp.start()
        cp.wait()


@jax.jit
def _run(hidden, tokens_2d):
    from jax.experimental.compute_on import compute_on
    with compute_on("tpu_sparsecore"):
        norms2_sc = _sc_norms(hidden)

    norms2_tc = pl.pallas_call(
        _norms_body,
        grid=(TC_GRID,),
        in_specs=[pl.BlockSpec((ROWS_PER_BLOCK, DIM), lambda i: (i, 0))],
        out_specs=pl.BlockSpec((SUBL, 128), lambda i: (i, 0)),
        out_shape=jax.ShapeDtypeStruct((TC_ROWS // 128, 128), jnp.float32),
        cost_estimate=pl.CostEstimate(
            flops=2 * TC_ROWS * DIM,
            bytes_accessed=TC_ROWS * DIM * 4,
            transcendentals=0,
        ),
    )(hidden)

    norms2 = jnp.concatenate(
        [norms2_tc, norms2_sc.reshape(SC_ROWS // 128, 128)], axis=0)

    anchors, meta = pl.pallas_call(
        _select_body,
        in_specs=[
            pl.BlockSpec(memory_space=pltpu.VMEM),
            pl.BlockSpec(memory_space=pltpu.VMEM),
            pl.BlockSpec(memory_space=pl.ANY),
        ],
        out_specs=[
            pl.BlockSpec(memory_space=pltpu.VMEM),
            pl.BlockSpec(memory_space=pltpu.VMEM),
        ],
        out_shape=[
            jax.ShapeDtypeStruct((K, DIM), jnp.float32),
            jax.ShapeDtypeStruct((8, 128), jnp.int32),
        ],
        scratch_shapes=[pltpu.SemaphoreType.DMA],
    )(norms2, tokens_2d, hidden)
    return anchors, meta


def kernel(hidden, tokens):
    tokens_2d = tokens.astype(jnp.int32).reshape(N // 128, 128)
    anchors, meta = _run(hidden, tokens_2d)
    sel_tokens = meta[0, :K].astype(tokens.dtype)
    return anchors, sel_tokens


# speedup vs baseline: 1.4366x; 1.4366x over previous
import jax
import jax.numpy as jnp
from jax import lax
from jax.experimental import pallas as pl
from jax.experimental.pallas import tpu as pltpu

N = 8192
DIM = 4096
K = 3

BLK = 512
GRID_F = N // BLK
SUB = BLK // 128


def _fused_body(h_blk, tokens_ref, hid_any, anchors_ref, meta_ref,
                norms_ref, sem):
    i = pl.program_id(0)
    x = h_blk[...]
    s = jnp.sum(x * x, axis=1)
    norms_ref[pl.ds(i * SUB, SUB), :] = s.reshape(SUB, 128)

    @pl.when(i == GRID_F - 1)
    def _():
        v = norms_ref[...]
        row = lax.broadcasted_iota(jnp.int32, v.shape, 0)
        lane = lax.broadcasted_iota(jnp.int32, v.shape, 1)
        gidx = row * 128 + lane
        big = jnp.int32(2**31 - 1)

        idxs = []
        for _ in range(K):
            m = jnp.max(v)
            cand = jnp.where(v == m, gidx, big)
            ik = jnp.min(cand)
            idxs.append(ik)
            v = jnp.where(gidx == ik, jnp.float32(-1.0), v)

        t = tokens_ref[...]
        toks = [jnp.sum(jnp.where(gidx == ik, t, 0)) for ik in idxs]

        lane8 = lax.broadcasted_iota(jnp.int32, (8, 128), 1)
        meta_ref[...] = jnp.where(lane8 == 0, toks[0],
                                  jnp.where(lane8 == 1, toks[1],
                                            jnp.where(lane8 == 2, toks[2], 0)))

        for k, ik in enumerate(idxs):
            cp = pltpu.make_async_copy(hid_any.at[pl.ds(ik, 1), :],
                                       anchors_ref.at[pl.ds(k, 1), :], sem)
            cp.start()
            cp.wait()


@jax.jit
def _run(hidden, tokens_2d):
    anchors, meta = pl.pallas_call(
        _fused_body,
        grid=(GRID_F,),
        in_specs=[
            pl.BlockSpec((BLK, DIM), lambda i: (i, 0)),
            pl.BlockSpec(memory_space=pltpu.VMEM),
            pl.BlockSpec(memory_space=pl.ANY),
        ],
        out_specs=[
            pl.BlockSpec((K, DIM), lambda i: (0, 0)),
            pl.BlockSpec((8, 128), lambda i: (0, 0)),
        ],
        out_shape=[
            jax.ShapeDtypeStruct((K, DIM), jnp.float32),
            jax.ShapeDtypeStruct((8, 128), jnp.int32),
        ],
        scratch_shapes=[
            pltpu.VMEM((N // 128, 128), jnp.float32),
            pltpu.SemaphoreType.DMA,
        ],
        compiler_params=pltpu.CompilerParams(
            dimension_semantics=("arbitrary",)),
    )(hidden, tokens_2d, hidden)
    return anchors, meta


def kernel(hidden, tokens):
    tokens_2d = tokens.astype(jnp.int32).reshape(N // 128, 128)
    anchors, meta = _run(hidden, tokens_2d)
    sel_tokens = meta[0, :K].astype(tokens.dtype)
    return anchors, sel_tokens
